# baseline (speedup 1.0000x reference)
import functools

import jax
import jax.numpy as jnp
from jax import lax
from jax.experimental import pallas as pl
from jax.experimental.pallas import tpu as pltpu
from jax.experimental.pallas import tpu_sc as plsc

N = 10000
D = 128
E = 320000

NC = 2
NS = 16
NW = NC * NS
CH = 128

E_TOT = E + N
NCH = -(-E_TOT // (NW * CH))
NCH += NCH % 2
EPW = NCH * CH
E_PAD = NW * EPW

NP = 10112
RPS = NP // NS

NCH0 = 162
NCH1 = 2 * NCH - NCH0
TOTCH = NS * (NCH0 + NCH1)

_MESH = plsc.VectorSubcoreMesh(
    core_axis_name="c", subcore_axis_name="s", num_cores=NC, num_subcores=NS
)


@functools.partial(
    pl.kernel,
    out_type=jax.ShapeDtypeStruct((NC, NP, D), jnp.float32),
    mesh=_MESH,
    scratch_types=[
        pltpu.VMEM((NCH, CH), jnp.int32),
        pltpu.VMEM((CH, D), jnp.float32),
        pltpu.VMEM_SHARED((NP, D), jnp.float32),
    ],
)
def _deg_kernel(dst_hbm, ones_hbm, zeros_hbm, out_hbm, dst_v, ones_v, acc_sh):
    c = lax.axis_index("c")
    s = lax.axis_index("s")
    w = s * NC + c
    pltpu.sync_copy(dst_hbm.at[w], dst_v)
    pltpu.sync_copy(ones_hbm, ones_v)
    pltpu.sync_copy(
        zeros_hbm.at[pl.ds(s * RPS, RPS)], acc_sh.at[pl.ds(s * RPS, RPS)]
    )
    plsc.subcore_barrier()

    def chunk(j, carry):
        pltpu.sync_copy(ones_v, acc_sh.at[dst_v.at[j]], add=True)
        return carry

    lax.fori_loop(0, NCH, chunk, 0)
    plsc.subcore_barrier()
    pltpu.sync_copy(
        acc_sh.at[pl.ds(s * RPS, RPS)], out_hbm.at[c, pl.ds(s * RPS, RPS)]
    )


@functools.partial(
    pl.kernel,
    out_type=jax.ShapeDtypeStruct((NC, NP, D), jnp.float32),
    mesh=_MESH,
    scratch_types=[
        pltpu.VMEM((CH,), jnp.int32),
        pltpu.VMEM((CH,), jnp.int32),
        pltpu.VMEM((CH,), jnp.int32),
        pltpu.VMEM((CH,), jnp.int32),
        pltpu.VMEM((CH, D), jnp.float32),
        pltpu.VMEM((CH, D), jnp.float32),
        pltpu.VMEM_SHARED((NP, D), jnp.float32),
        pltpu.SemaphoreType.DMA,
        pltpu.SemaphoreType.DMA,
        pltpu.SemaphoreType.DMA,
        pltpu.SemaphoreType.DMA,
        pltpu.SemaphoreType.DMA,
        pltpu.SemaphoreType.DMA,
    ],
)
def _scatter_kernel(
    h_hbm, src_hbm, dst_hbm, zeros_hbm, out_hbm,
    s0, s1, d0, d1, buf0, buf1, acc_sh,
    sis0, sis1, sid0, sid1, semg0, semg1,
):
    c = lax.axis_index("c")
    s = lax.axis_index("s")
    pltpu.sync_copy(
        zeros_hbm.at[pl.ds(s * RPS, RPS)], acc_sh.at[pl.ds(s * RPS, RPS)]
    )

    def run(base, nch):
        pltpu.async_copy(src_hbm.at[base], s0, sis0)
        pltpu.async_copy(dst_hbm.at[base], d0, sid0)
        pltpu.async_copy(src_hbm.at[base + 1], s1, sis1)
        pltpu.async_copy(dst_hbm.at[base + 1], d1, sid1)
        plsc.subcore_barrier()

        def chunk_pair(k, carry):
            j0 = base + 2 * k
            pltpu.make_async_copy(src_hbm.at[0], s0, sis0).wait()
            pltpu.async_copy(h_hbm.at[s0], buf0, semg0)
            pltpu.make_async_copy(src_hbm.at[0], s1, sis1).wait()
            pltpu.async_copy(h_hbm.at[s1], buf1, semg1)

            pltpu.make_async_copy(h_hbm.at[pl.ds(0, CH)], buf0, semg0).wait()
            pltpu.make_async_copy(dst_hbm.at[0], d0, sid0).wait()
            pltpu.sync_copy(buf0, acc_sh.at[d0], add=True)

            @pl.when(k < nch // 2 - 1)
            def _():
                pltpu.async_copy(src_hbm.at[j0 + 2], s0, sis0)
                pltpu.async_copy(dst_hbm.at[j0 + 2], d0, sid0)

            pltpu.make_async_copy(h_hbm.at[pl.ds(0, CH)], buf1, semg1).wait()
            pltpu.make_async_copy(dst_hbm.at[0], d1, sid1).wait()
            pltpu.sync_copy(buf1, acc_sh.at[d1], add=True)

            @pl.when(k < nch // 2 - 1)
            def _():
                pltpu.async_copy(src_hbm.at[j0 + 3], s1, sis1)
                pltpu.async_copy(dst_hbm.at[j0 + 3], d1, sid1)

            return carry

        lax.fori_loop(0, nch // 2, chunk_pair, 0)

    @pl.when(c == 0)
    def _():
        run(s * NCH0, NCH0)

    @pl.when(c == 1)
    def _():
        run(NS * NCH0 + s * NCH1, NCH1)

    plsc.subcore_barrier()
    pltpu.sync_copy(
        acc_sh.at[pl.ds(s * RPS, RPS)], out_hbm.at[c, pl.ds(s * RPS, RPS)]
    )


def _dinv_col(degp):
    deg = degp[0] + degp[1]
    return lax.rsqrt(jnp.maximum(deg, 1.0))


def _tc_a_body(degp_ref, xp_ref, w_ref, o_ref):
    h = jnp.dot(xp_ref[...], w_ref[...], preferred_element_type=jnp.float32)
    o_ref[...] = h * _dinv_col(degp_ref[...])


def _tc_b_body(degp_ref, accp_ref, b_ref, w_ref, o_ref):
    dinv = _dinv_col(degp_ref[...])
    g = accp_ref[0] + accp_ref[1]
    z = jnp.maximum(g * dinv + b_ref[...][None, :], 0.0)
    rows = lax.broadcasted_iota(jnp.int32, (NP, D), 0)
    z = jnp.where(rows < N, z, 0.0)
    h = jnp.dot(z, w_ref[...], preferred_element_type=jnp.float32)
    o_ref[...] = h * dinv


def _tc_c_body(degp_ref, accp_ref, b_ref, o_ref):
    dinv = _dinv_col(degp_ref[...])
    g = accp_ref[0] + accp_ref[1]
    o_ref[...] = (g * dinv)[:N] + b_ref[...][None, :]


_tc_a = pl.pallas_call(
    _tc_a_body, out_shape=jax.ShapeDtypeStruct((NP, D), jnp.float32)
)
_tc_b = pl.pallas_call(
    _tc_b_body, out_shape=jax.ShapeDtypeStruct((NP, D), jnp.float32)
)
_tc_c = pl.pallas_call(
    _tc_c_body, out_shape=jax.ShapeDtypeStruct((N, D), jnp.float32)
)


def kernel(x, edge_index, W1, b1, W2, b2):
    loops = jnp.arange(N, dtype=jnp.int32)
    pad = E_PAD - E_TOT
    src_flat = jnp.concatenate(
        [edge_index[0], loops, jnp.full((pad,), N, jnp.int32)]
    )
    dst_flat = jnp.concatenate(
        [edge_index[1], loops, jnp.full((pad,), N, jnp.int32)]
    )
    dst = dst_flat.reshape(NW, NCH, CH)
    src2 = src_flat.reshape(TOTCH, CH)
    dst2 = dst_flat.reshape(TOTCH, CH)

    onesD = jnp.ones((CH, D), jnp.float32)
    zerosD = jnp.zeros((NP, D), jnp.float32)
    xp = jnp.concatenate([x, jnp.zeros((NP - N, D), jnp.float32)], axis=0)

    degp = _deg_kernel(dst, onesD, zerosD)
    hs1 = _tc_a(degp, xp, W1)
    acc1 = _scatter_kernel(hs1, src2, dst2, zerosD)
    hs2 = _tc_b(degp, acc1, b1, W2)
    acc2 = _scatter_kernel(hs2, src2, dst2, zerosD)
    return _tc_c(degp, acc2, b2)


# speedup vs baseline: 1.1776x; 1.1776x over previous
import functools

import jax
import jax.numpy as jnp
from jax import lax
from jax.experimental import pallas as pl
from jax.experimental.pallas import tpu as pltpu
from jax.experimental.pallas import tpu_sc as plsc

N = 10000
D = 128
E = 320000

NC = 2
NS = 16
NW = NC * NS
CH = 128

E_TOT = E + N
NCH = -(-E_TOT // (NW * CH))
NCH += NCH % 2
EPW = NCH * CH
E_PAD = NW * EPW

NP = 10112
RPS = NP // NS

NCH0 = 132
NCH1 = 2 * NCH - NCH0
TOTCH = NS * (NCH0 + NCH1)

_MESH = plsc.VectorSubcoreMesh(
    core_axis_name="c", subcore_axis_name="s", num_cores=NC, num_subcores=NS
)


@functools.partial(
    pl.kernel,
    out_type=jax.ShapeDtypeStruct((NC, NP, D), jnp.float32),
    mesh=_MESH,
    scratch_types=[
        pltpu.VMEM((NCH, CH), jnp.int32),
        pltpu.VMEM((CH, D), jnp.float32),
        pltpu.VMEM_SHARED((NP, D), jnp.float32),
    ],
)
def _deg_kernel(dst_hbm, ones_hbm, zeros_hbm, out_hbm, dst_v, ones_v, acc_sh):
    c = lax.axis_index("c")
    s = lax.axis_index("s")
    w = s * NC + c
    pltpu.sync_copy(dst_hbm.at[w], dst_v)
    pltpu.sync_copy(ones_hbm, ones_v)
    pltpu.sync_copy(
        zeros_hbm.at[pl.ds(s * RPS, RPS)], acc_sh.at[pl.ds(s * RPS, RPS)]
    )
    plsc.subcore_barrier()

    def chunk(j, carry):
        pltpu.sync_copy(ones_v, acc_sh.at[dst_v.at[j]], add=True)
        return carry

    lax.fori_loop(0, NCH, chunk, 0)
    plsc.subcore_barrier()
    pltpu.sync_copy(
        acc_sh.at[pl.ds(s * RPS, RPS)], out_hbm.at[c, pl.ds(s * RPS, RPS)]
    )


@functools.partial(
    pl.kernel,
    out_type=jax.ShapeDtypeStruct((NC, NP, D), jnp.float32),
    mesh=_MESH,
    scratch_types=[
        pltpu.VMEM((CH,), jnp.int32),
        pltpu.VMEM((CH,), jnp.int32),
        pltpu.VMEM((CH,), jnp.int32),
        pltpu.VMEM((CH,), jnp.int32),
        pltpu.VMEM((CH, D), jnp.float32),
        pltpu.VMEM((CH, D), jnp.float32),
        pltpu.VMEM_SHARED((NP, D), jnp.float32),
        pltpu.SemaphoreType.DMA,
        pltpu.SemaphoreType.DMA,
        pltpu.SemaphoreType.DMA,
        pltpu.SemaphoreType.DMA,
        pltpu.SemaphoreType.DMA,
        pltpu.SemaphoreType.DMA,
    ],
)
def _scatter_kernel(
    h_hbm, src_hbm, dst_hbm, zeros_hbm, out_hbm,
    s0, s1, d0, d1, buf0, buf1, acc_sh,
    sis0, sis1, sid0, sid1, semg0, semg1,
):
    c = lax.axis_index("c")
    s = lax.axis_index("s")
    pltpu.sync_copy(
        zeros_hbm.at[pl.ds(s * RPS, RPS)], acc_sh.at[pl.ds(s * RPS, RPS)]
    )

    def run(base, nch):
        pltpu.async_copy(src_hbm.at[base], s0, sis0)
        pltpu.async_copy(dst_hbm.at[base], d0, sid0)
        pltpu.async_copy(src_hbm.at[base + 1], s1, sis1)
        pltpu.async_copy(dst_hbm.at[base + 1], d1, sid1)
        plsc.subcore_barrier()

        def chunk_pair(k, carry):
            j0 = base + 2 * k
            pltpu.make_async_copy(src_hbm.at[0], s0, sis0).wait()
            pltpu.async_copy(h_hbm.at[s0], buf0, semg0)
            pltpu.make_async_copy(src_hbm.at[0], s1, sis1).wait()
            pltpu.async_copy(h_hbm.at[s1], buf1, semg1)

            pltpu.make_async_copy(h_hbm.at[pl.ds(0, CH)], buf0, semg0).wait()
            pltpu.make_async_copy(dst_hbm.at[0], d0, sid0).wait()
            pltpu.sync_copy(buf0, acc_sh.at[d0], add=True)

            @pl.when(k < nch // 2 - 1)
            def _():
                pltpu.async_copy(src_hbm.at[j0 + 2], s0, sis0)
                pltpu.async_copy(dst_hbm.at[j0 + 2], d0, sid0)

            pltpu.make_async_copy(h_hbm.at[pl.ds(0, CH)], buf1, semg1).wait()
            pltpu.make_async_copy(dst_hbm.at[0], d1, sid1).wait()
            pltpu.sync_copy(buf1, acc_sh.at[d1], add=True)

            @pl.when(k < nch // 2 - 1)
            def _():
                pltpu.async_copy(src_hbm.at[j0 + 3], s1, sis1)
                pltpu.async_copy(dst_hbm.at[j0 + 3], d1, sid1)

            return carry

        lax.fori_loop(0, nch // 2, chunk_pair, 0)

    @pl.when(c == 0)
    def _():
        run(s * NCH0, NCH0)

    @pl.when(c == 1)
    def _():
        run(NS * NCH0 + s * NCH1, NCH1)

    plsc.subcore_barrier()
    pltpu.sync_copy(
        acc_sh.at[pl.ds(s * RPS, RPS)], out_hbm.at[c, pl.ds(s * RPS, RPS)]
    )


def _dinv_col(degp):
    deg = degp[0] + degp[1]
    return lax.rsqrt(jnp.maximum(deg, 1.0))


def _tc_a_body(degp_ref, xp_ref, w_ref, o_ref):
    h = jnp.dot(xp_ref[...], w_ref[...], preferred_element_type=jnp.float32)
    o_ref[...] = h * _dinv_col(degp_ref[...])


def _tc_b_body(degp_ref, accp_ref, b_ref, w_ref, o_ref):
    dinv = _dinv_col(degp_ref[...])
    g = accp_ref[0] + accp_ref[1]
    z = jnp.maximum(g * dinv + b_ref[...][None, :], 0.0)
    rows = lax.broadcasted_iota(jnp.int32, (NP, D), 0)
    z = jnp.where(rows < N, z, 0.0)
    h = jnp.dot(z, w_ref[...], preferred_element_type=jnp.float32)
    o_ref[...] = h * dinv


def _tc_c_body(degp_ref, accp_ref, b_ref, o_ref):
    dinv = _dinv_col(degp_ref[...])
    g = accp_ref[0] + accp_ref[1]
    o_ref[...] = (g * dinv)[:N] + b_ref[...][None, :]


_tc_a = pl.pallas_call(
    _tc_a_body, out_shape=jax.ShapeDtypeStruct((NP, D), jnp.float32)
)
_tc_b = pl.pallas_call(
    _tc_b_body, out_shape=jax.ShapeDtypeStruct((NP, D), jnp.float32)
)
_tc_c = pl.pallas_call(
    _tc_c_body, out_shape=jax.ShapeDtypeStruct((N, D), jnp.float32)
)


def kernel(x, edge_index, W1, b1, W2, b2):
    loops = jnp.arange(N, dtype=jnp.int32)
    pad = E_PAD - E_TOT
    src_flat = jnp.concatenate(
        [edge_index[0], loops, jnp.full((pad,), N, jnp.int32)]
    )
    dst_flat = jnp.concatenate(
        [edge_index[1], loops, jnp.full((pad,), N, jnp.int32)]
    )
    dst = dst_flat.reshape(NW, NCH, CH)
    src2 = src_flat.reshape(TOTCH, CH)
    dst2 = dst_flat.reshape(TOTCH, CH)

    onesD = jnp.ones((CH, D), jnp.float32)
    zerosD = jnp.zeros((NP, D), jnp.float32)
    xp = jnp.concatenate([x, jnp.zeros((NP - N, D), jnp.float32)], axis=0)

    degp = _deg_kernel(dst, onesD, zerosD)
    hs1 = _tc_a(degp, xp, W1)
    acc1 = _scatter_kernel(hs1, src2, dst2, zerosD)
    hs2 = _tc_b(degp, acc1, b1, W2)
    acc2 = _scatter_kernel(hs2, src2, dst2, zerosD)
    return _tc_c(degp, acc2, b2)
